# baseline (speedup 1.0000x reference)
import functools

import jax
import jax.numpy as jnp
from jax import lax
from jax.experimental import pallas as pl
from jax.experimental.pallas import tpu as pltpu
from jax.experimental.pallas import tpu_sc as plsc

T = 2048
D = 1024
E = 8
K = 2
F = 512
FS = 1024
ET = 2 * E
P = T * K
BLK = 512
GB = 256
NB = P // GB
G = NB + ET - 1
GP = 32

NEG = -1e30


def _routing_body(x_ref, gcat_ref, bias_ref, m_ref,
                  w_ref, inv_ref, eid_ref, tid_ref, lo_ref, hi_ref, fv_ref):
    x = x_ref[...]
    logits = jnp.dot(x, gcat_ref[...], preferred_element_type=jnp.float32)
    st = jax.nn.softmax(logits[:, :E], axis=-1)
    sv = jax.nn.softmax(logits[:, E:], axis=-1)
    m = m_ref[...]
    s_sel = jnp.where(m > 0, sv, st)
    b_sel = s_sel + jnp.where(m > 0, bias_ref[1:2, :], bias_ref[0:1, :])

    iota8 = lax.broadcasted_iota(jnp.int32, (T, E), 1)
    mx1 = jnp.max(b_sel, axis=1, keepdims=True)
    i1 = jnp.min(jnp.where(b_sel == mx1, iota8, E), axis=1, keepdims=True)
    w1 = jnp.sum(jnp.where(iota8 == i1, s_sel, 0.0), axis=1, keepdims=True)
    b2 = jnp.where(iota8 == i1, NEG, b_sel)
    mx2 = jnp.max(b2, axis=1, keepdims=True)
    i2 = jnp.min(jnp.where(b2 == mx2, iota8, E), axis=1, keepdims=True)
    w2 = jnp.sum(jnp.where(iota8 == i2, s_sel, 0.0), axis=1, keepdims=True)
    denom = w1 + w2
    w1 = w1 / denom
    w2 = w2 / denom

    mi = (m > 0).astype(jnp.int32) * E
    eid1 = i1 + mi
    eid2 = i2 + mi
    ids = jnp.concatenate([eid1, eid2], axis=0)
    wts = jnp.concatenate([w1, w2], axis=0)

    iota16 = lax.broadcasted_iota(jnp.int32, (P, ET), 1)
    oh = (ids == iota16).astype(jnp.float32)
    counts_i = jnp.sum((ids == iota16).astype(jnp.int32), axis=0,
                       keepdims=True)
    offs_i = jnp.sum((ids < iota16).astype(jnp.int32), axis=0,
                     keepdims=True)
    offs = offs_i.astype(jnp.float32)

    bi = lax.broadcasted_iota(jnp.int32, (BLK, BLK), 0)
    bj = lax.broadcasted_iota(jnp.int32, (BLK, BLK), 1)
    tri = (bj < bi).astype(jnp.bfloat16)

    base = offs
    ranks = []
    for b in range(P // BLK):
        ohb = oh[b * BLK:(b + 1) * BLK, :]
        csum = jnp.dot(tri, ohb.astype(jnp.bfloat16),
                       preferred_element_type=jnp.float32)
        ranks.append(jnp.sum(ohb * (csum + base), axis=1, keepdims=True))
        base = base + jnp.sum(ohb, axis=0, keepdims=True)
    rank = jnp.concatenate(ranks, axis=0)

    inv_ref[...] = rank.astype(jnp.int32)
    w_ref[...] = wts

    ends_i = offs_i + counts_i
    nonempty = counts_i > 0
    t_first = jnp.where(nonempty, offs_i // GB, 0)
    t_last = jnp.where(nonempty, (ends_i - 1) // GB, -1)
    span = jnp.where(nonempty, t_last - t_first + 1, 0)
    ir = lax.broadcasted_iota(jnp.int32, (ET, ET), 0)
    ic = lax.broadcasted_iota(jnp.int32, (ET, ET), 1)
    mlt = (ir < ic).astype(jnp.bfloat16)
    step_start = jnp.dot(span.astype(jnp.bfloat16), mlt,
                         preferred_element_type=jnp.float32).astype(jnp.int32)
    total = jnp.sum(span, axis=1, keepdims=True)

    sc = lax.broadcasted_iota(jnp.int32, (GP, ET), 0)
    er = lax.broadcasted_iota(jnp.int32, (GP, ET), 1)
    in_e = (sc >= step_start) & (sc < step_start + span)
    eid_s = jnp.sum(jnp.where(in_e, er, 0), axis=1, keepdims=True)
    tfst_s = jnp.sum(jnp.where(in_e, t_first, 0), axis=1, keepdims=True)
    sst_s = jnp.sum(jnp.where(in_e, step_start, 0), axis=1, keepdims=True)
    offs_s = jnp.sum(jnp.where(in_e, offs_i, 0), axis=1, keepdims=True)
    ends_s = jnp.sum(jnp.where(in_e, ends_i, 0), axis=1, keepdims=True)
    stepc = lax.broadcasted_iota(jnp.int32, (GP, 1), 0)
    valid = stepc < total
    tid_s = jnp.where(valid, tfst_s + stepc - sst_s, NB - 1)
    lo_s = jnp.where(valid, jnp.clip(offs_s - tid_s * GB, 0, GB), 0)
    hi_s = jnp.where(valid, jnp.clip(ends_s - tid_s * GB, 0, GB), 0)
    fv_s = jnp.concatenate(
        [jnp.ones((1, 1), jnp.int32),
         (tid_s[1:] != tid_s[:-1]).astype(jnp.int32)], axis=0)
    eid_ref[...] = eid_s
    tid_ref[...] = tid_s
    lo_ref[...] = lo_s
    hi_ref[...] = hi_s
    fv_ref[...] = fv_s


def _routing(x, gcat, bias, m):
    return pl.pallas_call(
        _routing_body,
        out_shape=[
            jax.ShapeDtypeStruct((P, 1), jnp.float32),
            jax.ShapeDtypeStruct((P, 1), jnp.int32),
        ] + [jax.ShapeDtypeStruct((GP, 1), jnp.int32)] * 5,
    )(x, gcat, bias, m)


_NW = 32
_DCH = 32
_PPW = P // _NW


@functools.lru_cache(maxsize=None)
def _get_dispatch():
    mesh = plsc.VectorSubcoreMesh(core_axis_name="c", subcore_axis_name="s")

    @functools.partial(
        pl.kernel,
        out_type=jax.ShapeDtypeStruct((P, D), jnp.float32),
        mesh=mesh,
        scratch_types=[
            pltpu.VMEM((_PPW,), jnp.int32),
            pltpu.VMEM((_DCH,), jnp.int32),
            pltpu.VMEM((_DCH,), jnp.int32),
            pltpu.VMEM((_DCH, D), jnp.float32),
            pltpu.VMEM((_DCH, D), jnp.float32),
            pltpu.SemaphoreType.DMA,
            pltpu.SemaphoreType.DMA,
        ],
    )
    def _dispatch(x_hbm, inv_hbm, xs_hbm, idxall, ia, ib, ra, rb, sa, sb):
        wid = lax.axis_index("s") * 2 + lax.axis_index("c")
        pltpu.sync_copy(inv_hbm.at[pl.ds(wid * _PPW, _PPW)], idxall)
        idxb = [ia, ib]
        rows = [ra, rb]
        sems = [sa, sb]
        prev = None
        for c in range(_PPW // _DCH):
            k = c % 2
            for j in range(_DCH // 16):
                idxb[k][pl.ds(j * 16, 16)] = idxall[pl.ds(c * _DCH + j * 16, 16)]
            pltpu.sync_copy(
                x_hbm.at[pl.ds((wid * _PPW + c * _DCH) % T, _DCH)], rows[k])
            if prev is not None:
                prev.wait()
            prev = pltpu.async_copy(rows[k], xs_hbm.at[idxb[k]], sems[k])
        prev.wait()

    return _dispatch


def _gmm_body(eid_r, tid_r, lo_r, hi_r, fv_r,
              xs_b, twg_b, twu_b, twd_b, vwg_b, vwu_b, vwd_b, y_b):
    i = pl.program_id(0)
    e = eid_r[i]

    is_text = e < E
    wg = jnp.where(is_text, twg_b[0].astype(jnp.bfloat16),
                   vwg_b[0].astype(jnp.bfloat16))
    wu = jnp.where(is_text, twu_b[0].astype(jnp.bfloat16),
                   vwu_b[0].astype(jnp.bfloat16))
    wd = jnp.where(is_text, twd_b[0].astype(jnp.bfloat16),
                   vwd_b[0].astype(jnp.bfloat16))
    x = xs_b[...].astype(jnp.bfloat16)
    g = jnp.dot(x, wg, preferred_element_type=jnp.float32)
    u = jnp.dot(x, wu, preferred_element_type=jnp.float32)
    h = (g * jax.nn.sigmoid(g)) * u
    yp = jnp.dot(h.astype(jnp.bfloat16), wd, preferred_element_type=jnp.float32)
    ri = lax.broadcasted_iota(jnp.int32, (GB, 1), 0)
    msk = (ri >= lo_r[i]) & (ri < hi_r[i])
    yp = jnp.where(msk, yp, 0.0)

    @pl.when(fv_r[i] == 1)
    def _():
        y_b[...] = yp

    @pl.when(fv_r[i] == 0)
    def _():
        y_b[...] += yp


def _gmm(eid, tid, lo, hi, fv, xs, twg, twu, twd, vwg, vwu, vwd):
    tmap = lambda i, e, t, l, h, f: (jnp.minimum(e[i], E - 1), 0, 0)
    vmap = lambda i, e, t, l, h, f: (jnp.maximum(e[i] - E, 0), 0, 0)
    grid_spec = pltpu.PrefetchScalarGridSpec(
        num_scalar_prefetch=5,
        grid=(G,),
        in_specs=[
            pl.BlockSpec((GB, D), lambda i, e, t, l, h, f: (t[i], 0)),
            pl.BlockSpec((1, D, F), tmap),
            pl.BlockSpec((1, D, F), tmap),
            pl.BlockSpec((1, F, D), tmap),
            pl.BlockSpec((1, D, F), vmap),
            pl.BlockSpec((1, D, F), vmap),
            pl.BlockSpec((1, F, D), vmap),
        ],
        out_specs=pl.BlockSpec((GB, D), lambda i, e, t, l, h, f: (t[i], 0)),
    )
    return pl.pallas_call(
        _gmm_body,
        grid_spec=grid_spec,
        out_shape=jax.ShapeDtypeStruct((P, D), jnp.float32),
    )(eid, tid, lo, hi, fv, xs, twg, twu, twd, vwg, vwu, vwd)


def _shared_body(x_b, wg_ref, wu_ref, wd_ref, o_b):
    x = x_b[...].astype(jnp.bfloat16)
    g = jnp.dot(x, wg_ref[...].astype(jnp.bfloat16),
                preferred_element_type=jnp.float32)
    u = jnp.dot(x, wu_ref[...].astype(jnp.bfloat16),
                preferred_element_type=jnp.float32)
    h = (g * jax.nn.sigmoid(g)) * u
    o_b[...] = jnp.dot(h.astype(jnp.bfloat16), wd_ref[...].astype(jnp.bfloat16),
                       preferred_element_type=jnp.float32)


def _shared_mlp(x, wg, wu, wd):
    nblk = 8
    rb = T // nblk
    return pl.pallas_call(
        _shared_body,
        grid=(nblk,),
        in_specs=[
            pl.BlockSpec((rb, D), lambda i: (i, 0)),
            pl.BlockSpec((D, FS), lambda i: (0, 0)),
            pl.BlockSpec((D, FS), lambda i: (0, 0)),
            pl.BlockSpec((FS, D), lambda i: (0, 0)),
        ],
        out_specs=pl.BlockSpec((rb, D), lambda i: (i, 0)),
        out_shape=jax.ShapeDtypeStruct((T, D), jnp.float32),
    )(x, wg, wu, wd)


_CCH = 16
_NCH = T // _NW // _CCH


@functools.lru_cache(maxsize=None)
def _get_combine():
    mesh = plsc.VectorSubcoreMesh(core_axis_name="c", subcore_axis_name="s")

    _TPW = T // _NW
    buf = lambda: [
        pltpu.VMEM((_CCH,), jnp.int32),
        pltpu.VMEM((_CCH,), jnp.int32),
        pltpu.VMEM((_CCH, D), jnp.float32),
        pltpu.VMEM((_CCH, D), jnp.float32),
        pltpu.VMEM((_CCH, D), jnp.float32),
        pltpu.SemaphoreType.DMA,
        pltpu.SemaphoreType.DMA,
        pltpu.SemaphoreType.DMA,
    ]

    @functools.partial(
        pl.kernel,
        out_type=jax.ShapeDtypeStruct((T, D), jnp.float32),
        mesh=mesh,
        scratch_types=[
            pltpu.VMEM((_TPW,), jnp.int32),
            pltpu.VMEM((_TPW,), jnp.int32),
            pltpu.VMEM((_TPW,), jnp.float32),
            pltpu.VMEM((_TPW,), jnp.float32),
        ] + buf() + buf(),
    )
    def _combine(y_hbm, sh_hbm, inv_hbm, w_hbm, out_hbm,
                 i0a, i1a, w0a, w1a, *scr):
        bufs = [scr[:8], scr[8:]]
        wid = lax.axis_index("s") * 2 + lax.axis_index("c")
        tbase = wid * _TPW
        pltpu.sync_copy(inv_hbm.at[pl.ds(tbase, _TPW)], i0a)
        pltpu.sync_copy(inv_hbm.at[pl.ds(T + tbase, _TPW)], i1a)
        pltpu.sync_copy(w_hbm.at[pl.ds(tbase, _TPW)], w0a)
        pltpu.sync_copy(w_hbm.at[pl.ds(T + tbase, _TPW)], w1a)

        def issue(c, B):
            idx0, idx1, a, b, sh, sa, sb, ss = B
            idx0[...] = i0a[pl.ds(c * _CCH, _CCH)]
            idx1[...] = i1a[pl.ds(c * _CCH, _CCH)]
            return (pltpu.async_copy(y_hbm.at[idx0], a, sa),
                    pltpu.async_copy(y_hbm.at[idx1], b, sb),
                    pltpu.async_copy(sh_hbm.at[pl.ds(tbase + c * _CCH, _CCH)],
                                     sh, ss))

        def drain(c, B, cps):
            idx0, idx1, a, b, sh, sa, sb, ss = B
            for cp in cps:
                cp.wait()
            for r in range(_CCH):
                wa = w0a[pl.ds(c * _CCH, 16)][r % 16]
                wb = w1a[pl.ds(c * _CCH, 16)][r % 16]

                def body(jc, _, r=r, wa=wa, wb=wb):
                    col = jc * 16
                    a[r, pl.ds(col, 16)] = (wa * a[r, pl.ds(col, 16)]
                                            + wb * b[r, pl.ds(col, 16)]
                                            + sh[r, pl.ds(col, 16)])
                    return 0
                lax.fori_loop(0, D // 16, body, 0, unroll=8)
            pltpu.sync_copy(a, out_hbm.at[pl.ds(tbase + c * _CCH, _CCH)])

        cps = issue(0, bufs[0])
        for c in range(_NCH):
            nxt = issue(c + 1, bufs[(c + 1) % 2]) if c + 1 < _NCH else None
            drain(c, bufs[c % 2], cps)
            cps = nxt

    return _combine


def kernel(hidden_states, visual_token_mask, e_score_correction_bias,
           text_gate_w, vision_gate_w,
           text_w_gate, text_w_up, text_w_down,
           vision_w_gate, vision_w_up, vision_w_down,
           shared_w_gate, shared_w_up, shared_w_down):
    orig_shape = hidden_states.shape
    x = hidden_states.reshape(-1, orig_shape[-1])
    gcat = jnp.concatenate([text_gate_w, vision_gate_w], axis=1)
    m = visual_token_mask.reshape(-1, 1).astype(jnp.float32)

    w, inv, eid, tid, lo, hi, fv = _routing(x, gcat, e_score_correction_bias, m)

    xs = _get_dispatch()(x, inv.reshape(P))

    y = _gmm(eid.reshape(GP), tid.reshape(GP), lo.reshape(GP),
             hi.reshape(GP), fv.reshape(GP), xs,
             text_w_gate, text_w_up, text_w_down,
             vision_w_gate, vision_w_up, vision_w_down)

    sh = _shared_mlp(x, shared_w_gate, shared_w_up, shared_w_down)

    out = _get_combine()(y, sh, inv.reshape(P), w.reshape(P))
    return out.reshape(orig_shape)


# speedup vs baseline: 1.0536x; 1.0536x over previous
import functools

import jax
import jax.numpy as jnp
from jax import lax
from jax.experimental import pallas as pl
from jax.experimental.pallas import tpu as pltpu
from jax.experimental.pallas import tpu_sc as plsc

T = 2048
D = 1024
E = 8
K = 2
F = 512
FS = 1024
ET = 2 * E
P = T * K
BLK = 512
GB = 256
NB = P // GB
G = NB + ET - 1
GP = 32

NEG = -1e30


def _routing_body(x_ref, gcat_ref, bias_ref, m_ref,
                  w_ref, inv_ref, eid_ref, tid_ref, lo_ref, hi_ref, fv_ref):
    x = x_ref[...]
    logits = jnp.dot(x, gcat_ref[...], preferred_element_type=jnp.float32)
    st = jax.nn.softmax(logits[:, :E], axis=-1)
    sv = jax.nn.softmax(logits[:, E:], axis=-1)
    m = m_ref[...]
    s_sel = jnp.where(m > 0, sv, st)
    b_sel = s_sel + jnp.where(m > 0, bias_ref[1:2, :], bias_ref[0:1, :])

    iota8 = lax.broadcasted_iota(jnp.int32, (T, E), 1)
    mx1 = jnp.max(b_sel, axis=1, keepdims=True)
    i1 = jnp.min(jnp.where(b_sel == mx1, iota8, E), axis=1, keepdims=True)
    w1 = jnp.sum(jnp.where(iota8 == i1, s_sel, 0.0), axis=1, keepdims=True)
    b2 = jnp.where(iota8 == i1, NEG, b_sel)
    mx2 = jnp.max(b2, axis=1, keepdims=True)
    i2 = jnp.min(jnp.where(b2 == mx2, iota8, E), axis=1, keepdims=True)
    w2 = jnp.sum(jnp.where(iota8 == i2, s_sel, 0.0), axis=1, keepdims=True)
    denom = w1 + w2
    w1 = w1 / denom
    w2 = w2 / denom

    mi = (m > 0).astype(jnp.int32) * E
    eid1 = i1 + mi
    eid2 = i2 + mi
    ids = jnp.concatenate([eid1, eid2], axis=0)
    wts = jnp.concatenate([w1, w2], axis=0)

    iota16 = lax.broadcasted_iota(jnp.int32, (P, ET), 1)
    oh = (ids == iota16).astype(jnp.float32)
    counts_i = jnp.sum((ids == iota16).astype(jnp.int32), axis=0,
                       keepdims=True)
    offs_i = jnp.sum((ids < iota16).astype(jnp.int32), axis=0,
                     keepdims=True)
    offs = offs_i.astype(jnp.float32)

    bi = lax.broadcasted_iota(jnp.int32, (BLK, BLK), 0)
    bj = lax.broadcasted_iota(jnp.int32, (BLK, BLK), 1)
    tri = (bj < bi).astype(jnp.bfloat16)

    base = offs
    ranks = []
    for b in range(P // BLK):
        ohb = oh[b * BLK:(b + 1) * BLK, :]
        csum = jnp.dot(tri, ohb.astype(jnp.bfloat16),
                       preferred_element_type=jnp.float32)
        ranks.append(jnp.sum(ohb * (csum + base), axis=1, keepdims=True))
        base = base + jnp.sum(ohb, axis=0, keepdims=True)
    rank = jnp.concatenate(ranks, axis=0)

    inv_ref[...] = rank.astype(jnp.int32)
    w_ref[...] = wts

    ends_i = offs_i + counts_i
    nonempty = counts_i > 0
    t_first = jnp.where(nonempty, offs_i // GB, 0)
    t_last = jnp.where(nonempty, (ends_i - 1) // GB, -1)
    span = jnp.where(nonempty, t_last - t_first + 1, 0)
    ir = lax.broadcasted_iota(jnp.int32, (ET, ET), 0)
    ic = lax.broadcasted_iota(jnp.int32, (ET, ET), 1)
    mlt = (ir < ic).astype(jnp.bfloat16)
    step_start = jnp.dot(span.astype(jnp.bfloat16), mlt,
                         preferred_element_type=jnp.float32).astype(jnp.int32)
    total = jnp.sum(span, axis=1, keepdims=True)

    sc = lax.broadcasted_iota(jnp.int32, (GP, ET), 0)
    er = lax.broadcasted_iota(jnp.int32, (GP, ET), 1)
    in_e = (sc >= step_start) & (sc < step_start + span)
    eid_s = jnp.sum(jnp.where(in_e, er, 0), axis=1, keepdims=True)
    tfst_s = jnp.sum(jnp.where(in_e, t_first, 0), axis=1, keepdims=True)
    sst_s = jnp.sum(jnp.where(in_e, step_start, 0), axis=1, keepdims=True)
    offs_s = jnp.sum(jnp.where(in_e, offs_i, 0), axis=1, keepdims=True)
    ends_s = jnp.sum(jnp.where(in_e, ends_i, 0), axis=1, keepdims=True)
    stepc = lax.broadcasted_iota(jnp.int32, (GP, 1), 0)
    valid = stepc < total
    tid_s = jnp.where(valid, tfst_s + stepc - sst_s, NB - 1)
    lo_s = jnp.where(valid, jnp.clip(offs_s - tid_s * GB, 0, GB), 0)
    hi_s = jnp.where(valid, jnp.clip(ends_s - tid_s * GB, 0, GB), 0)
    fv_s = jnp.concatenate(
        [jnp.ones((1, 1), jnp.int32),
         (tid_s[1:] != tid_s[:-1]).astype(jnp.int32)], axis=0)
    eid_ref[...] = eid_s
    tid_ref[...] = tid_s
    lo_ref[...] = lo_s
    hi_ref[...] = hi_s
    fv_ref[...] = fv_s


def _routing(x, gcat, bias, m):
    return pl.pallas_call(
        _routing_body,
        out_shape=[
            jax.ShapeDtypeStruct((P, 1), jnp.float32),
            jax.ShapeDtypeStruct((P, 1), jnp.int32),
        ] + [jax.ShapeDtypeStruct((GP, 1), jnp.int32)] * 5,
    )(x, gcat, bias, m)


_NW = 32
_DCH = 64


@functools.lru_cache(maxsize=None)
def _get_dispatch():
    mesh = plsc.VectorSubcoreMesh(core_axis_name="c", subcore_axis_name="s")

    @functools.partial(
        pl.kernel,
        out_type=jax.ShapeDtypeStruct((P, D), jnp.float32),
        mesh=mesh,
        scratch_types=[
            pltpu.VMEM((_DCH,), jnp.int32),
            pltpu.VMEM((_DCH, D), jnp.float32),
            pltpu.SemaphoreType.DMA,
        ],
    )
    def _dispatch(x_hbm, inv_hbm, xs_hbm, idx_v, rows_v, sem):
        wid = lax.axis_index("s") * 2 + lax.axis_index("c")
        for c in range(P // _NW // _DCH):
            base = wid * (P // _NW) + c * _DCH
            pltpu.sync_copy(inv_hbm.at[pl.ds(base, _DCH)], idx_v)
            pltpu.sync_copy(x_hbm.at[pl.ds(base % T, _DCH)], rows_v)
            pltpu.async_copy(rows_v, xs_hbm.at[idx_v], sem).wait()

    return _dispatch


def _gmm_body(eid_r, tid_r, lo_r, hi_r, fv_r,
              xs_b, twg_b, twu_b, twd_b, vwg_b, vwu_b, vwd_b, y_b):
    i = pl.program_id(0)
    e = eid_r[i]

    is_text = e < E
    wg = jnp.where(is_text, twg_b[0].astype(jnp.bfloat16),
                   vwg_b[0].astype(jnp.bfloat16))
    wu = jnp.where(is_text, twu_b[0].astype(jnp.bfloat16),
                   vwu_b[0].astype(jnp.bfloat16))
    wd = jnp.where(is_text, twd_b[0].astype(jnp.bfloat16),
                   vwd_b[0].astype(jnp.bfloat16))
    x = xs_b[...].astype(jnp.bfloat16)
    g = jnp.dot(x, wg, preferred_element_type=jnp.float32)
    u = jnp.dot(x, wu, preferred_element_type=jnp.float32)
    h = (g * jax.nn.sigmoid(g)) * u
    yp = jnp.dot(h.astype(jnp.bfloat16), wd, preferred_element_type=jnp.float32)
    ri = lax.broadcasted_iota(jnp.int32, (GB, 1), 0)
    msk = (ri >= lo_r[i]) & (ri < hi_r[i])
    yp = jnp.where(msk, yp, 0.0)

    @pl.when(fv_r[i] == 1)
    def _():
        y_b[...] = yp

    @pl.when(fv_r[i] == 0)
    def _():
        y_b[...] += yp


def _gmm(eid, tid, lo, hi, fv, xs, twg, twu, twd, vwg, vwu, vwd):
    tmap = lambda i, e, t, l, h, f: (jnp.minimum(e[i], E - 1), 0, 0)
    vmap = lambda i, e, t, l, h, f: (jnp.maximum(e[i] - E, 0), 0, 0)
    grid_spec = pltpu.PrefetchScalarGridSpec(
        num_scalar_prefetch=5,
        grid=(G,),
        in_specs=[
            pl.BlockSpec((GB, D), lambda i, e, t, l, h, f: (t[i], 0)),
            pl.BlockSpec((1, D, F), tmap),
            pl.BlockSpec((1, D, F), tmap),
            pl.BlockSpec((1, F, D), tmap),
            pl.BlockSpec((1, D, F), vmap),
            pl.BlockSpec((1, D, F), vmap),
            pl.BlockSpec((1, F, D), vmap),
        ],
        out_specs=pl.BlockSpec((GB, D), lambda i, e, t, l, h, f: (t[i], 0)),
    )
    return pl.pallas_call(
        _gmm_body,
        grid_spec=grid_spec,
        out_shape=jax.ShapeDtypeStruct((P, D), jnp.float32),
    )(eid, tid, lo, hi, fv, xs, twg, twu, twd, vwg, vwu, vwd)


def _shared_body(x_b, wg_ref, wu_ref, wd_ref, o_b):
    x = x_b[...].astype(jnp.bfloat16)
    g = jnp.dot(x, wg_ref[...].astype(jnp.bfloat16),
                preferred_element_type=jnp.float32)
    u = jnp.dot(x, wu_ref[...].astype(jnp.bfloat16),
                preferred_element_type=jnp.float32)
    h = (g * jax.nn.sigmoid(g)) * u
    o_b[...] = jnp.dot(h.astype(jnp.bfloat16), wd_ref[...].astype(jnp.bfloat16),
                       preferred_element_type=jnp.float32)


def _shared_mlp(x, wg, wu, wd):
    nblk = 8
    rb = T // nblk
    return pl.pallas_call(
        _shared_body,
        grid=(nblk,),
        in_specs=[
            pl.BlockSpec((rb, D), lambda i: (i, 0)),
            pl.BlockSpec((D, FS), lambda i: (0, 0)),
            pl.BlockSpec((D, FS), lambda i: (0, 0)),
            pl.BlockSpec((FS, D), lambda i: (0, 0)),
        ],
        out_specs=pl.BlockSpec((rb, D), lambda i: (i, 0)),
        out_shape=jax.ShapeDtypeStruct((T, D), jnp.float32),
    )(x, wg, wu, wd)


_CCH = 32


@functools.lru_cache(maxsize=None)
def _get_combine():
    mesh = plsc.VectorSubcoreMesh(core_axis_name="c", subcore_axis_name="s")

    @functools.partial(
        pl.kernel,
        out_type=jax.ShapeDtypeStruct((T, D), jnp.float32),
        mesh=mesh,
        scratch_types=[
            pltpu.VMEM((_CCH,), jnp.int32),
            pltpu.VMEM((_CCH,), jnp.int32),
            pltpu.VMEM((_CCH,), jnp.float32),
            pltpu.VMEM((_CCH,), jnp.float32),
            pltpu.VMEM((_CCH, D), jnp.float32),
            pltpu.VMEM((_CCH, D), jnp.float32),
            pltpu.VMEM((_CCH, D), jnp.float32),
            pltpu.SemaphoreType.DMA,
            pltpu.SemaphoreType.DMA,
        ],
    )
    def _combine(y_hbm, sh_hbm, inv_hbm, w_hbm, out_hbm,
                 i0, i1, w0, w1, a, b, sh, s1, s2):
        wid = lax.axis_index("s") * 2 + lax.axis_index("c")
        for c in range(T // _NW // _CCH):
            tb = wid * (T // _NW) + c * _CCH
            pltpu.sync_copy(inv_hbm.at[pl.ds(tb, _CCH)], i0)
            pltpu.sync_copy(inv_hbm.at[pl.ds(T + tb, _CCH)], i1)
            cp1 = pltpu.async_copy(y_hbm.at[i0], a, s1)
            cp2 = pltpu.async_copy(y_hbm.at[i1], b, s2)
            pltpu.sync_copy(w_hbm.at[pl.ds(tb, _CCH)], w0)
            pltpu.sync_copy(w_hbm.at[pl.ds(T + tb, _CCH)], w1)
            pltpu.sync_copy(sh_hbm.at[pl.ds(tb, _CCH)], sh)
            cp1.wait()
            cp2.wait()
            for r in range(_CCH):
                wa = w0[pl.ds((r // 16) * 16, 16)][r % 16]
                wb = w1[pl.ds((r // 16) * 16, 16)][r % 16]

                def body(jc, _, r=r, wa=wa, wb=wb):
                    col = jc * 16
                    a[r, pl.ds(col, 16)] = (wa * a[r, pl.ds(col, 16)]
                                            + wb * b[r, pl.ds(col, 16)]
                                            + sh[r, pl.ds(col, 16)])
                    return 0
                lax.fori_loop(0, D // 16, body, 0, unroll=4)
            pltpu.sync_copy(a, out_hbm.at[pl.ds(tb, _CCH)])

    return _combine


def kernel(hidden_states, visual_token_mask, e_score_correction_bias,
           text_gate_w, vision_gate_w,
           text_w_gate, text_w_up, text_w_down,
           vision_w_gate, vision_w_up, vision_w_down,
           shared_w_gate, shared_w_up, shared_w_down):
    orig_shape = hidden_states.shape
    x = hidden_states.reshape(-1, orig_shape[-1])
    gcat = jnp.concatenate([text_gate_w, vision_gate_w], axis=1)
    m = visual_token_mask.reshape(-1, 1).astype(jnp.float32)

    w, inv, eid, tid, lo, hi, fv = _routing(x, gcat, e_score_correction_bias, m)

    xs = _get_dispatch()(x, inv.reshape(P))

    y = _gmm(eid.reshape(GP), tid.reshape(GP), lo.reshape(GP),
             hi.reshape(GP), fv.reshape(GP), xs,
             text_w_gate, text_w_up, text_w_down,
             vision_w_gate, vision_w_up, vision_w_down)

    sh = _shared_mlp(x, shared_w_gate, shared_w_up, shared_w_down)

    out = _get_combine()(y, sh, inv.reshape(P), w.reshape(P))
    return out.reshape(orig_shape)
